# baseline (speedup 1.0000x reference)
import jax
import jax.numpy as jnp
from jax import lax
from jax.experimental import pallas as pl
from jax.experimental.pallas import tpu as pltpu
from jax.experimental.pallas import tpu_sc as plsc

_C = 3
_F = 5
_K = _F * _F
_HO = 48
_WO = 48
_HI = _HO + _F - 1
_WI = _WO + _F - 1
_NPIX = _HO * _WO
_NWORKERS = 32
_PPW = _NPIX // _NWORKERS
_NVEC = (_PPW + 15) // 16
_ROX = 0
_ROY = _K
_RMK = 2 * _K
_RVT = 3 * _K
_RHT = 3 * _K + _F
_NSTK = 3 * _K + 2 * _F


def _dsep_body(stk_hbm, inp_hbm, out_hbm, stk_v, inp_v, out_v, sem):
  wid = lax.axis_index("s") * 2 + lax.axis_index("c")
  base = wid * _PPW

  copies = [
      pltpu.async_copy(stk_hbm.at[:, pl.ds(base, _PPW)],
                       stk_v.at[:, pl.ds(0, _PPW)], sem),
      pltpu.async_copy(inp_hbm.at[0], inp_v, sem),
  ]
  for cp in copies:
    cp.wait()

  iota = lax.broadcasted_iota(jnp.int32, (16,), 0)
  zero = jnp.zeros((16,), jnp.float32)
  zero_i = jnp.zeros((16,), jnp.int32)

  for vec in range(_NVEC):
    lp = iota + (vec * 16)
    if (vec + 1) * 16 > _PPW:
      lp = jnp.minimum(lp, _PPW - 1)
    pix = lp + base
    h = lax.div(pix, _WO)
    w = pix - h * _WO
    h_f = h.astype(jnp.float32)
    w_f = w.astype(jnp.float32)
    cols16 = pl.ds(vec * 16, 16)
    hh_c = [stk_v[_RHT + fx, cols16] for fx in range(_F)]

    def body(fy, accs, lp=lp, h_f=h_f, w_f=w_f, hh_c=hh_c, cols16=cols16):
      accs = list(accs)
      vv = stk_v[fy + _RVT, cols16]
      fy_f = fy.astype(jnp.float32)
      k0 = fy * _F
      for fx in range(_F):
        k = k0 + fx
        ox = stk_v[k + _ROX, cols16]
        oy = stk_v[k + _ROY, cols16]
        mk = stk_v[k + _RMK, cols16]
        pos_x = oy + (w_f + float(fx - 1))
        pos_y = ox + (h_f + (fy_f - 1.0))
        pos_x = jnp.minimum(jnp.maximum(pos_x, 0.0), float(_WI - 1))
        pos_y = jnp.minimum(jnp.maximum(pos_y, 0.0), float(_HI - 1))
        left = jnp.minimum(jnp.maximum(pos_x.astype(jnp.int32), 0), _WI - 1)
        top = jnp.minimum(jnp.maximum(pos_y.astype(jnp.int32), 0), _HI - 1)
        fracx = pos_x - left.astype(jnp.float32)
        fracy = pos_y - top.astype(jnp.float32)
        row_t = top * _WI
        row_b = jnp.minimum(row_t + _WI, (_HI - 1) * _WI)
        i_tl = row_t + left
        i_tr = jnp.minimum(i_tl + 1, row_t + (_WI - 1))
        dx = i_tr - i_tl
        i_bl = row_b + left
        i_br = i_bl + dx
        wgt = vv * hh_c[fx] * mk
        par = fx & 1
        for c, off in zip(range(_C), (0, _HI * _WI, 2 * _HI * _WI)):
          tl = plsc.load_gather(inp_v, [i_tl + off])
          tr = plsc.load_gather(inp_v, [i_tr + off])
          bl = plsc.load_gather(inp_v, [i_bl + off])
          br = plsc.load_gather(inp_v, [i_br + off])
          top_l = tl + fracx * (tr - tl)
          bot_l = bl + fracx * (br - bl)
          val = top_l + fracy * (bot_l - top_l)
          accs[2 * c + par] = accs[2 * c + par] + val * wgt
      return tuple(accs)

    accs = lax.fori_loop(0, _F, body, (zero,) * (2 * _C))
    for c in range(_C):
      out_v[pl.ds(c * 80 + vec * 16, 16)] = accs[2 * c] + accs[2 * c + 1]

  for c in range(_C):
    pltpu.sync_copy(out_v.at[pl.ds(c * 80, _PPW)],
                    out_hbm.at[c, pl.ds(base, _PPW)])


@jax.jit
def _dsepconv_sc(stk, inp):
  mesh = plsc.VectorSubcoreMesh(core_axis_name="c", subcore_axis_name="s")
  run = pl.kernel(
      _dsep_body,
      out_type=jax.ShapeDtypeStruct((_C, _NPIX), jnp.float32),
      mesh=mesh,
      compiler_params=pltpu.CompilerParams(
          needs_layout_passes=False, use_tc_tiling_on_sc=False),
      scratch_types=[
          pltpu.VMEM((_NSTK, 80), jnp.float32),
          pltpu.VMEM((_C * _HI * _WI,), jnp.float32),
          pltpu.VMEM((_C * 80,), jnp.float32),
          pltpu.SemaphoreType.DMA,
      ],
  )
  return run(stk, inp)


def kernel(tensorInput, tensorVertical, tensorHorizontal, tensorOffsetX,
           tensorOffsetY, tensorMask):
  stk = jnp.concatenate([
      tensorOffsetX.reshape(_K, _NPIX),
      tensorOffsetY.reshape(_K, _NPIX),
      tensorMask.reshape(_K, _NPIX),
      tensorVertical.reshape(_F, _NPIX),
      tensorHorizontal.reshape(_F, _NPIX),
  ], axis=0)
  inp = tensorInput.reshape(1, _C * _HI * _WI)
  out = _dsepconv_sc(stk, inp)
  return out.reshape(1, _C, _HO, _WO)


# speedup vs baseline: 1.0849x; 1.0849x over previous
import jax
import jax.numpy as jnp
from jax import lax
from jax.experimental import pallas as pl
from jax.experimental.pallas import tpu as pltpu
from jax.experimental.pallas import tpu_sc as plsc

_C = 3
_F = 5
_K = _F * _F
_HO = 48
_WO = 48
_HI = _HO + _F - 1
_WI = _WO + _F - 1
_NPIX = _HO * _WO
_NWORKERS = 32
_PPW = _NPIX // _NWORKERS
_NVEC = (_PPW + 15) // 16
_ROX = 0
_ROY = _K
_RMK = 2 * _K
_RVT = 3 * _K
_RHT = 3 * _K + _F
_NSTK = 3 * _K + 2 * _F


def _dsep_body(stk_hbm, inp_hbm, out_hbm, stk_v, inp_v, out_v, sem):
  wid = lax.axis_index("s") * 2 + lax.axis_index("c")
  base = wid * _PPW

  copies = [
      pltpu.async_copy(stk_hbm.at[:, pl.ds(base, _PPW)],
                       stk_v.at[:, pl.ds(0, _PPW)], sem),
      pltpu.async_copy(inp_hbm.at[0], inp_v, sem),
  ]
  for cp in copies:
    cp.wait()

  iota = lax.broadcasted_iota(jnp.int32, (16,), 0)
  zero = jnp.zeros((16,), jnp.float32)
  zero_i = jnp.zeros((16,), jnp.int32)

  for vec in range(_NVEC):
    lp = iota + (vec * 16)
    if (vec + 1) * 16 > _PPW:
      lp = jnp.minimum(lp, _PPW - 1)
    pix = lp + base
    h = lax.div(pix, _WO)
    w = pix - h * _WO
    h_f = h.astype(jnp.float32)
    w_f = w.astype(jnp.float32)
    def body(k, accs, lp=lp, h_f=h_f, w_f=w_f):
      a0, a1, a2 = accs
      fy = lax.div(k, _F)
      fx = k - fy * _F
      kv = zero_i + k
      fyv = zero_i + fy
      ox = plsc.load_gather(stk_v, [kv + _ROX, lp])
      oy = plsc.load_gather(stk_v, [kv + _ROY, lp])
      mk = plsc.load_gather(stk_v, [kv + _RMK, lp])
      vv = plsc.load_gather(stk_v, [fyv + _RVT, lp])
      hh = plsc.load_gather(stk_v, [(kv - fyv * _F) + _RHT, lp])
      pos_x = oy + (w_f + (fx.astype(jnp.float32) - 1.0))
      pos_y = ox + (h_f + (fy.astype(jnp.float32) - 1.0))
      pos_x = jnp.minimum(jnp.maximum(pos_x, 0.0), float(_WI - 1))
      pos_y = jnp.minimum(jnp.maximum(pos_y, 0.0), float(_HI - 1))
      left = pos_x.astype(jnp.int32)
      top = pos_y.astype(jnp.int32)
      fracx = pos_x - left.astype(jnp.float32)
      fracy = pos_y - top.astype(jnp.float32)
      row_t = top * _WI
      row_b = jnp.minimum(row_t + _WI, (_HI - 1) * _WI)
      i_tl = row_t + left
      i_tr = jnp.minimum(i_tl + 1, row_t + (_WI - 1))
      dx = i_tr - i_tl
      i_bl = row_b + left
      i_br = i_bl + dx
      wgt = vv * hh * mk
      outs = []
      for acc, off in zip((a0, a1, a2), (0, _HI * _WI, 2 * _HI * _WI)):
        tl = plsc.load_gather(inp_v, [i_tl + off])
        tr = plsc.load_gather(inp_v, [i_tr + off])
        bl = plsc.load_gather(inp_v, [i_bl + off])
        br = plsc.load_gather(inp_v, [i_br + off])
        top_l = tl + fracx * (tr - tl)
        bot_l = bl + fracx * (br - bl)
        val = top_l + fracy * (bot_l - top_l)
        outs.append(acc + val * wgt)
      return tuple(outs)

    a0, a1, a2 = lax.fori_loop(0, _K, body, (zero, zero, zero))
    for c, acc in zip(range(_C), (a0, a1, a2)):
      out_v[pl.ds(c * 80 + vec * 16, 16)] = acc

  for c in range(_C):
    pltpu.sync_copy(out_v.at[pl.ds(c * 80, _PPW)],
                    out_hbm.at[c, pl.ds(base, _PPW)])


@jax.jit
def _dsepconv_sc(stk, inp):
  mesh = plsc.VectorSubcoreMesh(core_axis_name="c", subcore_axis_name="s")
  run = pl.kernel(
      _dsep_body,
      out_type=jax.ShapeDtypeStruct((_C, _NPIX), jnp.float32),
      mesh=mesh,
      compiler_params=pltpu.CompilerParams(
          needs_layout_passes=False, use_tc_tiling_on_sc=False),
      scratch_types=[
          pltpu.VMEM((_NSTK, 80), jnp.float32),
          pltpu.VMEM((_C * _HI * _WI,), jnp.float32),
          pltpu.VMEM((_C * 80,), jnp.float32),
          pltpu.SemaphoreType.DMA,
      ],
  )
  return run(stk, inp)


def kernel(tensorInput, tensorVertical, tensorHorizontal, tensorOffsetX,
           tensorOffsetY, tensorMask):
  stk = jnp.concatenate([
      tensorOffsetX.reshape(_K, _NPIX),
      tensorOffsetY.reshape(_K, _NPIX),
      tensorMask.reshape(_K, _NPIX),
      tensorVertical.reshape(_F, _NPIX),
      tensorHorizontal.reshape(_F, _NPIX),
  ], axis=0)
  inp = tensorInput.reshape(1, _C * _HI * _WI)
  out = _dsepconv_sc(stk, inp)
  return out.reshape(1, _C, _HO, _WO)


# speedup vs baseline: 1.1066x; 1.0199x over previous
import jax
import jax.numpy as jnp
from jax import lax
from jax.experimental import pallas as pl
from jax.experimental.pallas import tpu as pltpu
from jax.experimental.pallas import tpu_sc as plsc

_C = 3
_F = 5
_K = _F * _F
_HO = 48
_WO = 48
_HI = _HO + _F - 1
_WI = _WO + _F - 1
_NPIX = _HO * _WO
_NWORKERS = 32
_PPW = _NPIX // _NWORKERS
_NVEC = (_PPW + 15) // 16
_ROX = 0
_ROY = _K
_RMK = 2 * _K
_RVT = 3 * _K
_RHT = 3 * _K + _F
_NSTK = 3 * _K + 2 * _F


def _dsep_body(stk_hbm, inp_hbm, out_hbm, stk_v, inp_v, out_v, sem):
  wid = lax.axis_index("s") * 2 + lax.axis_index("c")
  base = wid * _PPW

  copies = [
      pltpu.async_copy(stk_hbm.at[:, pl.ds(base, _PPW)],
                       stk_v.at[:, pl.ds(0, _PPW)], sem),
      pltpu.async_copy(inp_hbm.at[0], inp_v, sem),
  ]
  for cp in copies:
    cp.wait()

  iota = lax.broadcasted_iota(jnp.int32, (16,), 0)
  zero = jnp.zeros((16,), jnp.float32)
  zero_i = jnp.zeros((16,), jnp.int32)

  def vec_body(vec, _):
    lp = jnp.minimum(iota + vec * 16, _PPW - 1)
    pix = lp + base
    h = lax.div(pix, _WO)
    w = pix - h * _WO
    h_f = h.astype(jnp.float32)
    w_f = w.astype(jnp.float32)
    def body(k, accs, lp=lp, h_f=h_f, w_f=w_f):
      a0, a1, a2 = accs
      fy = lax.div(k, _F)
      fx = k - fy * _F
      kv = zero_i + k
      fyv = zero_i + fy
      ox = plsc.load_gather(stk_v, [kv + _ROX, lp])
      oy = plsc.load_gather(stk_v, [kv + _ROY, lp])
      mk = plsc.load_gather(stk_v, [kv + _RMK, lp])
      vv = plsc.load_gather(stk_v, [fyv + _RVT, lp])
      hh = plsc.load_gather(stk_v, [(kv - fyv * _F) + _RHT, lp])
      pos_x = oy + (w_f + (fx.astype(jnp.float32) - 1.0))
      pos_y = ox + (h_f + (fy.astype(jnp.float32) - 1.0))
      pos_x = jnp.minimum(jnp.maximum(pos_x, 0.0), float(_WI - 1))
      pos_y = jnp.minimum(jnp.maximum(pos_y, 0.0), float(_HI - 1))
      left = pos_x.astype(jnp.int32)
      top = pos_y.astype(jnp.int32)
      fracx = pos_x - left.astype(jnp.float32)
      fracy = pos_y - top.astype(jnp.float32)
      row_t = top * _WI
      row_b = jnp.minimum(row_t + _WI, (_HI - 1) * _WI)
      i_tl = row_t + left
      i_tr = jnp.minimum(i_tl + 1, row_t + (_WI - 1))
      dx = i_tr - i_tl
      i_bl = row_b + left
      i_br = i_bl + dx
      wgt = vv * hh * mk
      outs = []
      for acc, off in zip((a0, a1, a2), (0, _HI * _WI, 2 * _HI * _WI)):
        tl = plsc.load_gather(inp_v, [i_tl + off])
        tr = plsc.load_gather(inp_v, [i_tr + off])
        bl = plsc.load_gather(inp_v, [i_bl + off])
        br = plsc.load_gather(inp_v, [i_br + off])
        top_l = tl + fracx * (tr - tl)
        bot_l = bl + fracx * (br - bl)
        val = top_l + fracy * (bot_l - top_l)
        outs.append(acc + val * wgt)
      return tuple(outs)

    a0, a1, a2 = lax.fori_loop(0, _K, body, (zero, zero, zero))
    for c, acc in zip(range(_C), (a0, a1, a2)):
      out_v[pl.ds(c * 80 + vec * 16, 16)] = acc
    return 0

  lax.fori_loop(0, _NVEC, vec_body, 0)

  for c in range(_C):
    pltpu.sync_copy(out_v.at[pl.ds(c * 80, _PPW)],
                    out_hbm.at[c, pl.ds(base, _PPW)])


@jax.jit
def _dsepconv_sc(stk, inp):
  mesh = plsc.VectorSubcoreMesh(core_axis_name="c", subcore_axis_name="s")
  run = pl.kernel(
      _dsep_body,
      out_type=jax.ShapeDtypeStruct((_C, _NPIX), jnp.float32),
      mesh=mesh,
      compiler_params=pltpu.CompilerParams(
          needs_layout_passes=False, use_tc_tiling_on_sc=False),
      scratch_types=[
          pltpu.VMEM((_NSTK, 80), jnp.float32),
          pltpu.VMEM((_C * _HI * _WI,), jnp.float32),
          pltpu.VMEM((_C * 80,), jnp.float32),
          pltpu.SemaphoreType.DMA,
      ],
  )
  return run(stk, inp)


def kernel(tensorInput, tensorVertical, tensorHorizontal, tensorOffsetX,
           tensorOffsetY, tensorMask):
  stk = jnp.concatenate([
      tensorOffsetX.reshape(_K, _NPIX),
      tensorOffsetY.reshape(_K, _NPIX),
      tensorMask.reshape(_K, _NPIX),
      tensorVertical.reshape(_F, _NPIX),
      tensorHorizontal.reshape(_F, _NPIX),
  ], axis=0)
  inp = tensorInput.reshape(1, _C * _HI * _WI)
  out = _dsepconv_sc(stk, inp)
  return out.reshape(1, _C, _HO, _WO)
